# Initial kernel scaffold; baseline (speedup 1.0000x reference)
import jax
import jax.numpy as jnp
from jax.experimental import pallas as pl


def kernel(x, psm_single, record_len, pairwise_t_matrix):
    raise NotImplementedError("write your pallas kernel here")


# speedup vs baseline: 19.4786x; 19.4786x over previous
import functools

import jax
import jax.numpy as jnp
import numpy as np
from jax.experimental import pallas as pl


def _gauss_coeffs(k_size=5, sigma=1.0):
    center = k_size // 2
    x, y = np.mgrid[0 - center:k_size - center, 0 - center:k_size - center]
    g = 1.0 / (2 * np.pi * sigma) * np.exp(-(np.square(x) + np.square(y)) / (2 * np.square(sigma)))
    return g.astype(np.float32)


def _mask_body(psm_ref, mask_ref, *, g, A, H, W, K, L):
    conf = jax.nn.sigmoid(psm_ref[0, 0])
    for a in range(1, A):
        conf = jnp.maximum(conf, jax.nn.sigmoid(psm_ref[0, a]))
    kh, kw = g.shape
    ph, pw = (kh - 1) // 2, (kw - 1) // 2
    zc = jnp.zeros((H, pw), jnp.float32)
    p = jnp.concatenate([zc, conf, zc], axis=1)
    zr = jnp.zeros((ph, W + 2 * pw), jnp.float32)
    p = jnp.concatenate([zr, p, zr], axis=0)
    acc = jnp.zeros((H, W), jnp.float32)
    for dy in range(kh):
        for dx in range(kw):
            acc = acc + float(g[dy, dx]) * p[dy:dy + H, dx:dx + W]
    keys = jax.lax.bitcast_convert_type(acc, jnp.int32)

    def body(_, lohi):
        lo, hi = lohi
        mid = hi - ((hi - lo) >> 1)
        cnt = jnp.sum((keys >= mid).astype(jnp.int32))
        big = cnt >= K
        return jnp.where(big, mid, lo), jnp.where(big, hi, mid - 1)

    lo, _ = jax.lax.fori_loop(
        0, 31, body, (jnp.int32(0), jnp.int32(2**31 - 1)))
    m = (keys >= lo).astype(jnp.float32)
    i = pl.program_id(0)
    m = jnp.where(i % L == 0, jnp.ones_like(m), m)
    mask_ref[0, 0] = m


def _fusion_body(x_ref, m_ref, o_ref, *, L, C):
    isc = float(1.0 / np.sqrt(C))
    x0 = x_ref[0, 0]
    s = []
    for m in range(L):
        d = jnp.sum(x0 * x_ref[0, m], axis=0)
        s.append(m_ref[0, m] * d * isc)
    smax = s[0]
    for m in range(1, L):
        smax = jnp.maximum(smax, s[m])
    e = [jnp.exp(sm - smax) for sm in s]
    den = e[0]
    for m in range(1, L):
        den = den + e[m]
    inv_den = 1.0 / den
    acc = (e[0] * inv_den)[None] * x0
    for m in range(1, L):
        w = e[m] * m_ref[0, m] * inv_den
        acc = acc + w[None] * x_ref[0, m]
    o_ref[0] = acc


def kernel(x, psm_single, record_len, pairwise_t_matrix):
    N, C, H, W = x.shape
    B = record_len.shape[0]
    L = N // B
    A = psm_single.shape[1]
    K = (H * W) // 2
    HT = 8
    g = _gauss_coeffs(5, 1.0)

    mask = pl.pallas_call(
        functools.partial(_mask_body, g=g, A=A, H=H, W=W, K=K, L=L),
        grid=(N,),
        in_specs=[pl.BlockSpec((1, A, H, W), lambda i: (i, 0, 0, 0))],
        out_specs=pl.BlockSpec((1, 1, H, W), lambda i: (i // L, i % L, 0, 0)),
        out_shape=jax.ShapeDtypeStruct((B, L, H, W), jnp.float32),
    )(psm_single)

    xs = x.reshape(B, L, C, H, W)
    fused = pl.pallas_call(
        functools.partial(_fusion_body, L=L, C=C),
        grid=(B, H // HT),
        in_specs=[
            pl.BlockSpec((1, L, C, HT, W), lambda b, t: (b, 0, 0, t, 0)),
            pl.BlockSpec((1, L, HT, W), lambda b, t: (b, 0, t, 0)),
        ],
        out_specs=pl.BlockSpec((1, C, HT, W), lambda b, t: (b, 0, t, 0)),
        out_shape=jax.ShapeDtypeStruct((B, C, H, W), jnp.float32),
    )(xs, mask)

    rate = jnp.float32(K / (H * W))
    return fused, rate
